# baseline (speedup 1.0000x reference)
import jax
import jax.numpy as jnp
from jax import lax
from jax.experimental import pallas as pl
from jax.experimental.pallas import tpu as pltpu
from jax.experimental.pallas import tpu_sc as plsc

N = 10000
E = 160000
D_IN = 256
D_HID = 256
D_OUT = 64

F1 = 176
F2 = 128
SB = 32
SB2 = 48
PB1 = 36
PB2 = 36
NSUB = 16
NCORE = 2
EPAD = 165888
NBTOT = EPAD // SB
ROWS_PT = N // NSUB


def _sc_spmm(z0, z1, srcb, dstb, wb, F, split_edges):
  nvec = F // 16
  nb32 = F // 32
  tail = (F % 32) // 16
  PW = (nb32 + tail) * 16
  if split_edges:
    SBr, PB = SB2, PB2
    nb = EPAD // SB2 // (2 * NSUB)
  else:
    SBr, PB = SB, PB1
    nb = NBTOT // NSUB
  nphase = nb // PB

  def body(z0_hbm, z1_hbm, src_hbm, dst_hbm, w_hbm, out0_hbm, out1_hbm,
           acc, pbuf, fbuf, ebuf, gs0, gs1, ss0, ss1):
    cid = lax.axis_index("c")
    sid = lax.axis_index("s")

    def run(z_hbm, out_hbm):
      if split_edges:
        bbase = (cid * NSUB + sid) * nb
      else:
        bbase = sid * nb
      gsem = (gs0, gs1)
      ssem = (ss0, ss1)

      def zrow(j, _):
        for c in range(nvec):
          fbuf[0, j, pl.ds(c * 16, 16)] = jnp.zeros((16,), jnp.float32)
        return 0
      lax.fori_loop(0, SBr, zrow, 0)
      nz = ROWS_PT // SBr
      def zcopy(zi, _):
        pltpu.sync_copy(fbuf.at[0],
                        acc.at[pl.ds(sid * ROWS_PT + zi * SBr, SBr)])
        return 0
      lax.fori_loop(0, nz, zcopy, 0)
      rem = ROWS_PT - nz * SBr
      if rem:
        pltpu.sync_copy(fbuf.at[0, pl.ds(0, rem)],
                        acc.at[pl.ds(sid * ROWS_PT + nz * SBr, rem)])
      plsc.subcore_barrier()

      def g_desc(k, x):
        return pltpu.make_async_copy(z_hbm.at[ebuf.at[0, k]], pbuf.at[x],
                                     gsem[x])

      def s_desc(k, x):
        return pltpu.make_async_copy(fbuf.at[x], acc.at[ebuf.at[1, k]],
                                     ssem[x])

      def s_start(k, x):
        pltpu.async_copy(fbuf.at[x], acc.at[ebuf.at[1, k]], ssem[x],
                         add=True)

      def scale(k, x):
        def gbody(g, _):
          wvec = plsc.bitcast(ebuf[2, k, pl.ds(g * 16, 16)], jnp.float32)
          dnums = lax.GatherDimensionNumbers(
              offset_dims=(), collapsed_slice_dims=(0,), start_index_map=(0,))
          for j in range(16):
            wj = lax.gather(wvec, jnp.full((16, 1), j, jnp.int32),
                            dimension_numbers=dnums, slice_sizes=(1,),
                            mode=lax.GatherScatterMode.PROMISE_IN_BOUNDS)
            r = g * 16 + j
            for b in range(nb32 + tail):
              vi = plsc.bitcast(pbuf[x, r, pl.ds(16 * b, 16)], jnp.int32)
              lo = plsc.bitcast(vi << 16, jnp.float32)
              fbuf[x, r, pl.ds(32 * b, 16)] = lo * wj
              if b < nb32:
                hi = plsc.bitcast(vi & jnp.int32(-65536), jnp.float32)
                fbuf[x, r, pl.ds(32 * b + 16, 16)] = hi * wj
          return 0
        lax.fori_loop(0, SBr // 16, gbody, 0)

      def phase(p, _):
        @pl.when(p > 0)
        def _():
          for x in (0, 1):
            s_desc(0, x).wait()
        bsl = pl.ds(bbase + p * PB, PB)
        pltpu.sync_copy(src_hbm.at[bsl], ebuf.at[0])
        pltpu.sync_copy(dst_hbm.at[bsl], ebuf.at[1])
        pltpu.sync_copy(w_hbm.at[bsl], ebuf.at[2])
        g_desc(0, 0).start()

        def step(t, _):
          for x in (0, 1):
            k = 2 * t + x
            y = 1 - x
            @pl.when(jnp.logical_and(k >= 2, k <= PB - 1))
            def _():
              s_desc(0, x).wait()
            @pl.when(k <= PB - 2)
            def _():
              g_desc(k + 1, y).start()
            g_desc(k, x).wait()
            scale(k, x)
            s_start(k, x)
          return 0
        lax.fori_loop(0, PB // 2, step, 0)
        return 0
      lax.fori_loop(0, nphase, phase, 0)
      for x in (0, 1):
        s_desc(0, x).wait()
      plsc.subcore_barrier()

      sl = pl.ds(sid * ROWS_PT, ROWS_PT)
      pltpu.sync_copy(acc.at[sl], out_hbm.at[sl])

    @pl.when(cid == 0)
    def _():
      run(z0_hbm, out0_hbm)

    @pl.when(cid == 1)
    def _():
      run(z1_hbm, out1_hbm)

  mesh = plsc.VectorSubcoreMesh(core_axis_name="c", subcore_axis_name="s")
  f = pl.kernel(
      body,
      out_type=[jax.ShapeDtypeStruct((N, F), jnp.float32),
                jax.ShapeDtypeStruct((N, F), jnp.float32)],
      mesh=mesh,
      scratch_types=[
          pltpu.VMEM_SHARED((N, F), jnp.float32),
          pltpu.VMEM((2, SBr, PW), jnp.float32),
          pltpu.VMEM((2, SBr, F), jnp.float32),
          pltpu.VMEM((3, PB, SBr), jnp.int32),
          pltpu.SemaphoreType.DMA,
          pltpu.SemaphoreType.DMA,
          pltpu.SemaphoreType.DMA,
          pltpu.SemaphoreType.DMA,
      ],
      compiler_params=pltpu.CompilerParams(use_tc_tiling_on_sc=False,
                                           needs_layout_passes=False),
  )
  return f(z0, z1, srcb, dstb, wb)


def _pack_bf16(z, F):
  nb32 = F // 32
  main = z[:, :nb32 * 32].reshape(N, nb32, 2, 16).transpose(0, 1, 3, 2)
  main = main.reshape(N, nb32 * 32)
  parts = [main]
  if F % 32:
    t = jnp.stack([z[:, nb32 * 32:], jnp.zeros((N, 16), jnp.float32)],
                  axis=-1).reshape(N, 32)
    parts.append(t)
  stored = jnp.concatenate(parts, axis=1).astype(jnp.bfloat16)
  pw = stored.shape[1] // 2
  return lax.bitcast_convert_type(stored.reshape(N, pw, 2), jnp.float32)


def _prep_body(x_ref, w1_ref, y_ref, src_ref, dst_ref, ew_ref,
               z0_ref, z1_ref, *sd_ref):
  xw = jnp.dot(x_ref[...], w1_ref[...], preferred_element_type=jnp.float32)
  z0_ref[...] = xw[:, :F1]
  r = xw.shape[0]
  ones = jnp.ones((r, 1), jnp.float32)
  zeros = jnp.zeros((r, F1 - (D_HID - F1) - D_OUT - 1), jnp.float32)
  z1_ref[...] = jnp.concatenate([xw[:, F1:], y_ref[...], ones, zeros], axis=1)

  @pl.when(pl.program_id(0) == 0)
  def _():
    pad = EPAD - E
    ri = lax.broadcasted_iota(jnp.int32, (pad // SB, SB), 0)
    ci = lax.broadcasted_iota(jnp.int32, (pad // SB, SB), 1)
    ar = ((ri * SB + ci) * 16) % N
    sd_ref[0][...] = jnp.concatenate([src_ref[...], ar], axis=0)
    sd_ref[1][...] = jnp.concatenate([dst_ref[...], ar], axis=0)
    wbits = lax.bitcast_convert_type(jnp.exp(ew_ref[...]), jnp.int32)
    zpad = jnp.zeros((pad // SB, SB), jnp.int32)
    sd_ref[2][...] = jnp.concatenate([wbits, zpad], axis=0)


def _mid_body(p0_ref, p1_ref, w2_ref, b1_ref, z2_ref):
  dn = p1_ref[:, D_HID - F1 + D_OUT:D_HID - F1 + D_OUT + 1] + 1e-16
  pre = jnp.concatenate([p0_ref[...], p1_ref[:, :D_HID - F1]], axis=1)
  h = jnp.maximum(pre / dn + b1_ref[...], 0.0)
  hw2 = jnp.dot(h, w2_ref[...], preferred_element_type=jnp.float32)
  z2_ref[...] = jnp.concatenate(
      [hw2, p1_ref[:, D_HID - F1:D_HID - F1 + D_OUT] / dn], axis=1)


def _log_softmax(o):
  o = o - jnp.max(o, axis=1, keepdims=True)
  return o - jnp.log(jnp.sum(jnp.exp(o), axis=1, keepdims=True))


def _final_body(pa_ref, pb_ref, dn_ref, b2_ref, out_ref, y_ref):
  dn = dn_ref[:, D_HID - F1 + D_OUT:D_HID - F1 + D_OUT + 1] + 1e-16
  p2 = pa_ref[...] + pb_ref[...]
  out_ref[...] = _log_softmax(p2[:, :D_OUT] / dn + b2_ref[...])
  y_ref[...] = _log_softmax(p2[:, D_OUT:] / dn)


def kernel(X, adj, Y, W1, b1, W2, b2, edge_weight):
  src = adj[0]
  dst = adj[1]

  R = 1000
  grid = (N // R,)

  z10, z11, srcb, dstb, wb = pl.pallas_call(
      _prep_body,
      grid=grid,
      in_specs=[
          pl.BlockSpec((R, D_IN), lambda i: (i, 0)),
          pl.BlockSpec((D_IN, D_HID), lambda i: (0, 0)),
          pl.BlockSpec((R, D_OUT), lambda i: (i, 0)),
          pl.BlockSpec((E // SB, SB), lambda i: (0, 0)),
          pl.BlockSpec((E // SB, SB), lambda i: (0, 0)),
          pl.BlockSpec((E // SB, SB), lambda i: (0, 0)),
      ],
      out_specs=[
          pl.BlockSpec((R, F1), lambda i: (i, 0)),
          pl.BlockSpec((R, F1), lambda i: (i, 0)),
          pl.BlockSpec((NBTOT, SB), lambda i: (0, 0)),
          pl.BlockSpec((NBTOT, SB), lambda i: (0, 0)),
          pl.BlockSpec((NBTOT, SB), lambda i: (0, 0)),
      ],
      out_shape=[jax.ShapeDtypeStruct((N, F1), jnp.float32),
                 jax.ShapeDtypeStruct((N, F1), jnp.float32),
                 jax.ShapeDtypeStruct((NBTOT, SB), jnp.int32),
                 jax.ShapeDtypeStruct((NBTOT, SB), jnp.int32),
                 jax.ShapeDtypeStruct((NBTOT, SB), jnp.int32)],
  )(X, W1, Y, src.reshape(E // SB, SB), dst.reshape(E // SB, SB),
    edge_weight.reshape(E // SB, SB))

  p10, p11 = _sc_spmm(_pack_bf16(z10, F1), _pack_bf16(z11, F1),
                      srcb, dstb, wb, F1, split_edges=False)

  z2 = pl.pallas_call(
      _mid_body,
      grid=grid,
      in_specs=[
          pl.BlockSpec((R, F1), lambda i: (i, 0)),
          pl.BlockSpec((R, F1), lambda i: (i, 0)),
          pl.BlockSpec((D_HID, D_OUT), lambda i: (0, 0)),
          pl.BlockSpec((1, D_HID), lambda i: (0, 0)),
      ],
      out_specs=pl.BlockSpec((R, F2), lambda i: (i, 0)),
      out_shape=jax.ShapeDtypeStruct((N, F2), jnp.float32),
  )(p10, p11, W2, b1.reshape(1, D_HID))

  z2p = _pack_bf16(z2, F2)
  sb2 = srcb.reshape(EPAD // SB2, SB2)
  db2 = dstb.reshape(EPAD // SB2, SB2)
  wb2 = wb.reshape(EPAD // SB2, SB2)
  p2a, p2b = _sc_spmm(z2p, z2p, sb2, db2, wb2, F2, split_edges=True)

  out, y2 = pl.pallas_call(
      _final_body,
      grid=grid,
      in_specs=[
          pl.BlockSpec((R, F2), lambda i: (i, 0)),
          pl.BlockSpec((R, F2), lambda i: (i, 0)),
          pl.BlockSpec((R, F1), lambda i: (i, 0)),
          pl.BlockSpec((1, D_OUT), lambda i: (0, 0)),
      ],
      out_specs=[
          pl.BlockSpec((R, D_OUT), lambda i: (i, 0)),
          pl.BlockSpec((R, D_OUT), lambda i: (i, 0)),
      ],
      out_shape=[jax.ShapeDtypeStruct((N, D_OUT), jnp.float32),
                 jax.ShapeDtypeStruct((N, D_OUT), jnp.float32)],
  )(p2a, p2b, p11, b2.reshape(1, D_OUT))

  return (out, y2)


# speedup vs baseline: 1.8106x; 1.8106x over previous
import jax
import jax.numpy as jnp
from jax import lax
from jax.experimental import pallas as pl
from jax.experimental.pallas import tpu as pltpu
from jax.experimental.pallas import tpu_sc as plsc

N = 10000
E = 160000
D_IN = 256
D_HID = 256
D_OUT = 64

F1 = 176
F2 = 128
SB = 32
SB2 = 48
PB1 = 36
PB2 = 36
NSUB = 16
NCORE = 2
EPAD = 165888
NBTOT = EPAD // SB
ROWS_PT = N // NSUB


def _sc_spmm(z0, z1, srcb, dstb, wb, F, split_edges):
  nvec = F // 16
  if split_edges:
    SBr, PB = SB2, PB2
    nb = EPAD // SB2 // (2 * NSUB)
  else:
    SBr, PB = SB, PB1
    nb = NBTOT // NSUB
  nphase = nb // PB

  def body(z0_hbm, z1_hbm, src_hbm, dst_hbm, w_hbm, out0_hbm, out1_hbm,
           acc, rows, ebuf, gs0, gs1, gs2, ss0, ss1, ss2):
    cid = lax.axis_index("c")
    sid = lax.axis_index("s")

    def run(z_hbm, out_hbm):
      if split_edges:
        bbase = (cid * NSUB + sid) * nb
      else:
        bbase = sid * nb
      gsem = (gs0, gs1, gs2)
      ssem = (ss0, ss1, ss2)

      @plsc.parallel_loop(0, SBr)
      def _(j):
        for c in range(nvec):
          rows[0, j, pl.ds(c * 16, 16)] = jnp.zeros((16,), jnp.float32)
      nz = ROWS_PT // SBr
      def zcopy(zi, _):
        pltpu.sync_copy(rows.at[0],
                        acc.at[pl.ds(sid * ROWS_PT + zi * SBr, SBr)])
        return 0
      lax.fori_loop(0, nz, zcopy, 0)
      rem = ROWS_PT - nz * SBr
      if rem:
        pltpu.sync_copy(rows.at[0, pl.ds(0, rem)],
                        acc.at[pl.ds(sid * ROWS_PT + nz * SBr, rem)])
      plsc.subcore_barrier()

      def g_desc(k, x):
        return pltpu.make_async_copy(z_hbm.at[ebuf.at[0, k]], rows.at[x],
                                     gsem[x])

      def s_desc(k, x):
        return pltpu.make_async_copy(rows.at[x], acc.at[ebuf.at[1, k]],
                                     ssem[x])

      def s_start(k, x):
        pltpu.async_copy(rows.at[x], acc.at[ebuf.at[1, k]], ssem[x],
                         add=True)

      def scale(k, x):
        dnums = lax.GatherDimensionNumbers(
            offset_dims=(), collapsed_slice_dims=(0,), start_index_map=(0,))
        @plsc.parallel_loop(0, SBr // 16, unroll=2)
        def _(g):
          wvec = plsc.bitcast(ebuf[2, k, pl.ds(g * 16, 16)], jnp.float32)
          for j in range(16):
            wj = lax.gather(wvec, jnp.full((16, 1), j, jnp.int32),
                            dimension_numbers=dnums, slice_sizes=(1,),
                            mode=lax.GatherScatterMode.PROMISE_IN_BOUNDS)
            r = g * 16 + j
            for c in range(nvec):
              sl = pl.ds(c * 16, 16)
              rows[x, r, sl] = rows[x, r, sl] * wj

      def phase(p, _):
        @pl.when(p > 0)
        def _():
          for x in (0, 1, 2):
            s_desc(0, x).wait()
        bsl = pl.ds(bbase + p * PB, PB)
        pltpu.sync_copy(src_hbm.at[bsl], ebuf.at[0])
        pltpu.sync_copy(dst_hbm.at[bsl], ebuf.at[1])
        pltpu.sync_copy(w_hbm.at[bsl], ebuf.at[2])
        g_desc(0, 0).start()

        def step(t, _):
          for x in (0, 1, 2):
            k = 3 * t + x
            y = (x + 1) % 3
            @pl.when(jnp.logical_and(k >= 2, k <= PB - 2))
            def _():
              s_desc(0, y).wait()
            @pl.when(k <= PB - 2)
            def _():
              g_desc(k + 1, y).start()
            g_desc(k, x).wait()
            scale(k, x)
            s_start(k, x)
          return 0
        lax.fori_loop(0, PB // 3, step, 0)
        return 0
      lax.fori_loop(0, nphase, phase, 0)
      for x in (0, 1, 2):
        s_desc(0, x).wait()
      plsc.subcore_barrier()

      sl = pl.ds(sid * ROWS_PT, ROWS_PT)
      pltpu.sync_copy(acc.at[sl], out_hbm.at[sl])

    @pl.when(cid == 0)
    def _():
      run(z0_hbm, out0_hbm)

    @pl.when(cid == 1)
    def _():
      run(z1_hbm, out1_hbm)

  mesh = plsc.VectorSubcoreMesh(core_axis_name="c", subcore_axis_name="s")
  f = pl.kernel(
      body,
      out_type=[jax.ShapeDtypeStruct((N, F), jnp.float32),
                jax.ShapeDtypeStruct((N, F), jnp.float32)],
      mesh=mesh,
      scratch_types=[
          pltpu.VMEM_SHARED((N, F), jnp.float32),
          pltpu.VMEM((3, SBr, F), jnp.float32),
          pltpu.VMEM((3, PB, SBr), jnp.int32),
          pltpu.SemaphoreType.DMA,
          pltpu.SemaphoreType.DMA,
          pltpu.SemaphoreType.DMA,
          pltpu.SemaphoreType.DMA,
          pltpu.SemaphoreType.DMA,
          pltpu.SemaphoreType.DMA,
      ],
      compiler_params=pltpu.CompilerParams(use_tc_tiling_on_sc=False,
                                           needs_layout_passes=False),
  )
  return f(z0, z1, srcb, dstb, wb)


def _prep_body(x_ref, w1_ref, y_ref, src_ref, dst_ref, ew_ref,
               z0_ref, z1_ref, *sd_ref):
  xw = jnp.dot(x_ref[...], w1_ref[...], preferred_element_type=jnp.float32)
  z0_ref[...] = xw[:, :F1]
  r = xw.shape[0]
  ones = jnp.ones((r, 1), jnp.float32)
  zeros = jnp.zeros((r, F1 - (D_HID - F1) - D_OUT - 1), jnp.float32)
  z1_ref[...] = jnp.concatenate([xw[:, F1:], y_ref[...], ones, zeros], axis=1)

  @pl.when(pl.program_id(0) == 0)
  def _():
    pad = EPAD - E
    ri = lax.broadcasted_iota(jnp.int32, (pad // SB, SB), 0)
    ci = lax.broadcasted_iota(jnp.int32, (pad // SB, SB), 1)
    ar = ((ri * SB + ci) * 16) % N
    sd_ref[0][...] = jnp.concatenate([src_ref[...], ar], axis=0)
    sd_ref[1][...] = jnp.concatenate([dst_ref[...], ar], axis=0)
    wbits = lax.bitcast_convert_type(jnp.exp(ew_ref[...]), jnp.int32)
    zpad = jnp.zeros((pad // SB, SB), jnp.int32)
    sd_ref[2][...] = jnp.concatenate([wbits, zpad], axis=0)


def _mid_body(p0_ref, p1_ref, w2_ref, b1_ref, z2_ref):
  dn = p1_ref[:, D_HID - F1 + D_OUT:D_HID - F1 + D_OUT + 1] + 1e-16
  pre = jnp.concatenate([p0_ref[...], p1_ref[:, :D_HID - F1]], axis=1)
  h = jnp.maximum(pre / dn + b1_ref[...], 0.0)
  hw2 = jnp.dot(h, w2_ref[...], preferred_element_type=jnp.float32)
  z2_ref[...] = jnp.concatenate(
      [hw2, p1_ref[:, D_HID - F1:D_HID - F1 + D_OUT] / dn], axis=1)


def _log_softmax(o):
  o = o - jnp.max(o, axis=1, keepdims=True)
  return o - jnp.log(jnp.sum(jnp.exp(o), axis=1, keepdims=True))


def _final_body(pa_ref, pb_ref, dn_ref, b2_ref, out_ref, y_ref):
  dn = dn_ref[:, D_HID - F1 + D_OUT:D_HID - F1 + D_OUT + 1] + 1e-16
  p2 = pa_ref[...] + pb_ref[...]
  out_ref[...] = _log_softmax(p2[:, :D_OUT] / dn + b2_ref[...])
  y_ref[...] = _log_softmax(p2[:, D_OUT:] / dn)


def kernel(X, adj, Y, W1, b1, W2, b2, edge_weight):
  src = adj[0]
  dst = adj[1]

  R = 1000
  grid = (N // R,)

  z10, z11, srcb, dstb, wb = pl.pallas_call(
      _prep_body,
      grid=grid,
      in_specs=[
          pl.BlockSpec((R, D_IN), lambda i: (i, 0)),
          pl.BlockSpec((D_IN, D_HID), lambda i: (0, 0)),
          pl.BlockSpec((R, D_OUT), lambda i: (i, 0)),
          pl.BlockSpec((E // SB, SB), lambda i: (0, 0)),
          pl.BlockSpec((E // SB, SB), lambda i: (0, 0)),
          pl.BlockSpec((E // SB, SB), lambda i: (0, 0)),
      ],
      out_specs=[
          pl.BlockSpec((R, F1), lambda i: (i, 0)),
          pl.BlockSpec((R, F1), lambda i: (i, 0)),
          pl.BlockSpec((NBTOT, SB), lambda i: (0, 0)),
          pl.BlockSpec((NBTOT, SB), lambda i: (0, 0)),
          pl.BlockSpec((NBTOT, SB), lambda i: (0, 0)),
      ],
      out_shape=[jax.ShapeDtypeStruct((N, F1), jnp.float32),
                 jax.ShapeDtypeStruct((N, F1), jnp.float32),
                 jax.ShapeDtypeStruct((NBTOT, SB), jnp.int32),
                 jax.ShapeDtypeStruct((NBTOT, SB), jnp.int32),
                 jax.ShapeDtypeStruct((NBTOT, SB), jnp.int32)],
  )(X, W1, Y, src.reshape(E // SB, SB), dst.reshape(E // SB, SB),
    edge_weight.reshape(E // SB, SB))

  p10, p11 = _sc_spmm(z10, z11, srcb, dstb, wb, F1, split_edges=False)

  z2 = pl.pallas_call(
      _mid_body,
      grid=grid,
      in_specs=[
          pl.BlockSpec((R, F1), lambda i: (i, 0)),
          pl.BlockSpec((R, F1), lambda i: (i, 0)),
          pl.BlockSpec((D_HID, D_OUT), lambda i: (0, 0)),
          pl.BlockSpec((1, D_HID), lambda i: (0, 0)),
      ],
      out_specs=pl.BlockSpec((R, F2), lambda i: (i, 0)),
      out_shape=jax.ShapeDtypeStruct((N, F2), jnp.float32),
  )(p10, p11, W2, b1.reshape(1, D_HID))

  sb2 = srcb.reshape(EPAD // SB2, SB2)
  db2 = dstb.reshape(EPAD // SB2, SB2)
  wb2 = wb.reshape(EPAD // SB2, SB2)
  p2a, p2b = _sc_spmm(z2, z2, sb2, db2, wb2, F2, split_edges=True)

  out, y2 = pl.pallas_call(
      _final_body,
      grid=grid,
      in_specs=[
          pl.BlockSpec((R, F2), lambda i: (i, 0)),
          pl.BlockSpec((R, F2), lambda i: (i, 0)),
          pl.BlockSpec((R, F1), lambda i: (i, 0)),
          pl.BlockSpec((1, D_OUT), lambda i: (0, 0)),
      ],
      out_specs=[
          pl.BlockSpec((R, D_OUT), lambda i: (i, 0)),
          pl.BlockSpec((R, D_OUT), lambda i: (i, 0)),
      ],
      out_shape=[jax.ShapeDtypeStruct((N, D_OUT), jnp.float32),
                 jax.ShapeDtypeStruct((N, D_OUT), jnp.float32)],
  )(p2a, p2b, p11, b2.reshape(1, D_OUT))

  return (out, y2)


# speedup vs baseline: 1.8545x; 1.0242x over previous
import jax
import jax.numpy as jnp
from jax import lax
from jax.experimental import pallas as pl
from jax.experimental.pallas import tpu as pltpu
from jax.experimental.pallas import tpu_sc as plsc

N = 10000
E = 160000
D_IN = 256
D_HID = 256
D_OUT = 64

F1 = 176
F2 = 128
SB = 32
SB2 = 96
PB1 = 36
PB2 = 18
NSUB = 16
NCORE = 2
EPAD = 165888
NBTOT = EPAD // SB
ROWS_PT = N // NSUB


def _sc_spmm(z0, z1, srcb, dstb, wb, F, split_edges):
  nvec = F // 16
  if split_edges:
    SBr, PB = SB2, PB2
    nb = EPAD // SB2 // (2 * NSUB)
  else:
    SBr, PB = SB, PB1
    nb = NBTOT // NSUB
  nphase = nb // PB

  def body(z0_hbm, z1_hbm, src_hbm, dst_hbm, w_hbm, out0_hbm, out1_hbm,
           acc, rows, ebuf, gs0, gs1, gs2, ss0, ss1, ss2):
    cid = lax.axis_index("c")
    sid = lax.axis_index("s")

    def run(z_hbm, out_hbm):
      if split_edges:
        bbase = (cid * NSUB + sid) * nb
      else:
        bbase = sid * nb
      gsem = (gs0, gs1, gs2)
      ssem = (ss0, ss1, ss2)

      @plsc.parallel_loop(0, SBr)
      def _(j):
        for c in range(nvec):
          rows[0, j, pl.ds(c * 16, 16)] = jnp.zeros((16,), jnp.float32)
      nz = ROWS_PT // SBr
      def zcopy(zi, _):
        pltpu.sync_copy(rows.at[0],
                        acc.at[pl.ds(sid * ROWS_PT + zi * SBr, SBr)])
        return 0
      lax.fori_loop(0, nz, zcopy, 0)
      rem = ROWS_PT - nz * SBr
      if rem:
        pltpu.sync_copy(rows.at[0, pl.ds(0, rem)],
                        acc.at[pl.ds(sid * ROWS_PT + nz * SBr, rem)])
      plsc.subcore_barrier()

      def g_desc(k, x):
        return pltpu.make_async_copy(z_hbm.at[ebuf.at[0, k]], rows.at[x],
                                     gsem[x])

      def s_desc(k, x):
        return pltpu.make_async_copy(rows.at[x], acc.at[ebuf.at[1, k]],
                                     ssem[x])

      def s_start(k, x):
        pltpu.async_copy(rows.at[x], acc.at[ebuf.at[1, k]], ssem[x],
                         add=True)

      def scale(k, x):
        dnums = lax.GatherDimensionNumbers(
            offset_dims=(), collapsed_slice_dims=(0,), start_index_map=(0,))
        @plsc.parallel_loop(0, SBr // 16, unroll=2)
        def _(g):
          wvec = plsc.bitcast(ebuf[2, k, pl.ds(g * 16, 16)], jnp.float32)
          for j in range(16):
            wj = lax.gather(wvec, jnp.full((16, 1), j, jnp.int32),
                            dimension_numbers=dnums, slice_sizes=(1,),
                            mode=lax.GatherScatterMode.PROMISE_IN_BOUNDS)
            r = g * 16 + j
            for c in range(nvec):
              sl = pl.ds(c * 16, 16)
              rows[x, r, sl] = rows[x, r, sl] * wj

      def phase(p, _):
        @pl.when(p > 0)
        def _():
          for x in (0, 1, 2):
            s_desc(0, x).wait()
        bsl = pl.ds(bbase + p * PB, PB)
        pltpu.sync_copy(src_hbm.at[bsl], ebuf.at[0])
        pltpu.sync_copy(dst_hbm.at[bsl], ebuf.at[1])
        pltpu.sync_copy(w_hbm.at[bsl], ebuf.at[2])
        g_desc(0, 0).start()

        def step(t, _):
          for x in (0, 1, 2):
            k = 3 * t + x
            y = (x + 1) % 3
            @pl.when(jnp.logical_and(k >= 2, k <= PB - 2))
            def _():
              s_desc(0, y).wait()
            @pl.when(k <= PB - 2)
            def _():
              g_desc(k + 1, y).start()
            g_desc(k, x).wait()
            scale(k, x)
            s_start(k, x)
          return 0
        lax.fori_loop(0, PB // 3, step, 0)
        return 0
      lax.fori_loop(0, nphase, phase, 0)
      for x in (0, 1, 2):
        s_desc(0, x).wait()
      plsc.subcore_barrier()

      sl = pl.ds(sid * ROWS_PT, ROWS_PT)
      pltpu.sync_copy(acc.at[sl], out_hbm.at[sl])

    @pl.when(cid == 0)
    def _():
      run(z0_hbm, out0_hbm)

    @pl.when(cid == 1)
    def _():
      run(z1_hbm, out1_hbm)

  mesh = plsc.VectorSubcoreMesh(core_axis_name="c", subcore_axis_name="s")
  f = pl.kernel(
      body,
      out_type=[jax.ShapeDtypeStruct((N, F), jnp.float32),
                jax.ShapeDtypeStruct((N, F), jnp.float32)],
      mesh=mesh,
      scratch_types=[
          pltpu.VMEM_SHARED((N, F), jnp.float32),
          pltpu.VMEM((3, SBr, F), jnp.float32),
          pltpu.VMEM((3, PB, SBr), jnp.int32),
          pltpu.SemaphoreType.DMA,
          pltpu.SemaphoreType.DMA,
          pltpu.SemaphoreType.DMA,
          pltpu.SemaphoreType.DMA,
          pltpu.SemaphoreType.DMA,
          pltpu.SemaphoreType.DMA,
      ],
      compiler_params=pltpu.CompilerParams(use_tc_tiling_on_sc=False,
                                           needs_layout_passes=False),
  )
  return f(z0, z1, srcb, dstb, wb)


def _prep_body(x_ref, w1_ref, y_ref, src_ref, dst_ref, ew_ref,
               z0_ref, z1_ref, *sd_ref):
  xw = jnp.dot(x_ref[...], w1_ref[...], preferred_element_type=jnp.float32)
  z0_ref[...] = xw[:, :F1]
  r = xw.shape[0]
  ones = jnp.ones((r, 1), jnp.float32)
  zeros = jnp.zeros((r, F1 - (D_HID - F1) - D_OUT - 1), jnp.float32)
  z1_ref[...] = jnp.concatenate([xw[:, F1:], y_ref[...], ones, zeros], axis=1)

  @pl.when(pl.program_id(0) == 0)
  def _():
    pad = EPAD - E
    ri = lax.broadcasted_iota(jnp.int32, (pad // SB, SB), 0)
    ci = lax.broadcasted_iota(jnp.int32, (pad // SB, SB), 1)
    ar = ((ri * SB + ci) * 16) % N
    sd_ref[0][...] = jnp.concatenate([src_ref[...], ar], axis=0)
    sd_ref[1][...] = jnp.concatenate([dst_ref[...], ar], axis=0)
    wbits = lax.bitcast_convert_type(jnp.exp(ew_ref[...]), jnp.int32)
    zpad = jnp.zeros((pad // SB, SB), jnp.int32)
    sd_ref[2][...] = jnp.concatenate([wbits, zpad], axis=0)


def _mid_body(p0_ref, p1_ref, w2_ref, b1_ref, z2_ref):
  dn = p1_ref[:, D_HID - F1 + D_OUT:D_HID - F1 + D_OUT + 1] + 1e-16
  pre = jnp.concatenate([p0_ref[...], p1_ref[:, :D_HID - F1]], axis=1)
  h = jnp.maximum(pre / dn + b1_ref[...], 0.0)
  hw2 = jnp.dot(h, w2_ref[...], preferred_element_type=jnp.float32)
  z2_ref[...] = jnp.concatenate(
      [hw2, p1_ref[:, D_HID - F1:D_HID - F1 + D_OUT] / dn], axis=1)


def _log_softmax(o):
  o = o - jnp.max(o, axis=1, keepdims=True)
  return o - jnp.log(jnp.sum(jnp.exp(o), axis=1, keepdims=True))


def _final_body(pa_ref, pb_ref, dn_ref, b2_ref, out_ref, y_ref):
  dn = dn_ref[:, D_HID - F1 + D_OUT:D_HID - F1 + D_OUT + 1] + 1e-16
  p2 = pa_ref[...] + pb_ref[...]
  out_ref[...] = _log_softmax(p2[:, :D_OUT] / dn + b2_ref[...])
  y_ref[...] = _log_softmax(p2[:, D_OUT:] / dn)


def kernel(X, adj, Y, W1, b1, W2, b2, edge_weight):
  src = adj[0]
  dst = adj[1]

  R = 1000
  grid = (N // R,)

  z10, z11, srcb, dstb, wb = pl.pallas_call(
      _prep_body,
      grid=grid,
      in_specs=[
          pl.BlockSpec((R, D_IN), lambda i: (i, 0)),
          pl.BlockSpec((D_IN, D_HID), lambda i: (0, 0)),
          pl.BlockSpec((R, D_OUT), lambda i: (i, 0)),
          pl.BlockSpec((E // SB, SB), lambda i: (0, 0)),
          pl.BlockSpec((E // SB, SB), lambda i: (0, 0)),
          pl.BlockSpec((E // SB, SB), lambda i: (0, 0)),
      ],
      out_specs=[
          pl.BlockSpec((R, F1), lambda i: (i, 0)),
          pl.BlockSpec((R, F1), lambda i: (i, 0)),
          pl.BlockSpec((NBTOT, SB), lambda i: (0, 0)),
          pl.BlockSpec((NBTOT, SB), lambda i: (0, 0)),
          pl.BlockSpec((NBTOT, SB), lambda i: (0, 0)),
      ],
      out_shape=[jax.ShapeDtypeStruct((N, F1), jnp.float32),
                 jax.ShapeDtypeStruct((N, F1), jnp.float32),
                 jax.ShapeDtypeStruct((NBTOT, SB), jnp.int32),
                 jax.ShapeDtypeStruct((NBTOT, SB), jnp.int32),
                 jax.ShapeDtypeStruct((NBTOT, SB), jnp.int32)],
  )(X, W1, Y, src.reshape(E // SB, SB), dst.reshape(E // SB, SB),
    edge_weight.reshape(E // SB, SB))

  p10, p11 = _sc_spmm(z10, z11, srcb, dstb, wb, F1, split_edges=False)

  z2 = pl.pallas_call(
      _mid_body,
      grid=grid,
      in_specs=[
          pl.BlockSpec((R, F1), lambda i: (i, 0)),
          pl.BlockSpec((R, F1), lambda i: (i, 0)),
          pl.BlockSpec((D_HID, D_OUT), lambda i: (0, 0)),
          pl.BlockSpec((1, D_HID), lambda i: (0, 0)),
      ],
      out_specs=pl.BlockSpec((R, F2), lambda i: (i, 0)),
      out_shape=jax.ShapeDtypeStruct((N, F2), jnp.float32),
  )(p10, p11, W2, b1.reshape(1, D_HID))

  sb2 = srcb.reshape(EPAD // SB2, SB2)
  db2 = dstb.reshape(EPAD // SB2, SB2)
  wb2 = wb.reshape(EPAD // SB2, SB2)
  p2a, p2b = _sc_spmm(z2, z2, sb2, db2, wb2, F2, split_edges=True)

  out, y2 = pl.pallas_call(
      _final_body,
      grid=grid,
      in_specs=[
          pl.BlockSpec((R, F2), lambda i: (i, 0)),
          pl.BlockSpec((R, F2), lambda i: (i, 0)),
          pl.BlockSpec((R, F1), lambda i: (i, 0)),
          pl.BlockSpec((1, D_OUT), lambda i: (0, 0)),
      ],
      out_specs=[
          pl.BlockSpec((R, D_OUT), lambda i: (i, 0)),
          pl.BlockSpec((R, D_OUT), lambda i: (i, 0)),
      ],
      out_shape=[jax.ShapeDtypeStruct((N, D_OUT), jnp.float32),
                 jax.ShapeDtypeStruct((N, D_OUT), jnp.float32)],
  )(p2a, p2b, p11, b2.reshape(1, D_OUT))

  return (out, y2)


# speedup vs baseline: 1.9080x; 1.0289x over previous
import jax
import jax.numpy as jnp
from jax import lax
from jax.experimental import pallas as pl
from jax.experimental.pallas import tpu as pltpu
from jax.experimental.pallas import tpu_sc as plsc

N = 10000
E = 160000
D_IN = 256
D_HID = 256
D_OUT = 64

F1 = 176
F2 = 128
SB = 32
SB2 = 96
PB1 = 36
PB2 = 18
NSUB = 16
NCORE = 2
EPAD = 165888
NBTOT = EPAD // SB
ROWS_PT = N // NSUB


def _sc_spmm(z0, z1, srcb, dstb, wb, F, split_edges):
  nvec = F // 16
  if split_edges:
    SBr, PB = SB2, PB2
    nb = EPAD // SB2 // (2 * NSUB)
  else:
    SBr, PB = SB, PB1
    nb = NBTOT // NSUB
  nphase = nb // PB

  def body(z0_hbm, z1_hbm, src_hbm, dst_hbm, w_hbm, out0_hbm, out1_hbm,
           acc, rows, ebuf, gs0, gs1, gs2, ss0, ss1, ss2, isem):
    cid = lax.axis_index("c")
    sid = lax.axis_index("s")

    def run(z_hbm, out_hbm):
      if split_edges:
        bbase = (cid * NSUB + sid) * nb
      else:
        bbase = sid * nb
      gsem = (gs0, gs1, gs2)
      ssem = (ss0, ss1, ss2)

      @plsc.parallel_loop(0, SBr)
      def _(j):
        for c in range(nvec):
          rows[0, j, pl.ds(c * 16, 16)] = jnp.zeros((16,), jnp.float32)
      nz = ROWS_PT // SBr
      def zcopy(zi, _):
        pltpu.sync_copy(rows.at[0],
                        acc.at[pl.ds(sid * ROWS_PT + zi * SBr, SBr)])
        return 0
      lax.fori_loop(0, nz, zcopy, 0)
      rem = ROWS_PT - nz * SBr
      if rem:
        pltpu.sync_copy(rows.at[0, pl.ds(0, rem)],
                        acc.at[pl.ds(sid * ROWS_PT + nz * SBr, rem)])
      plsc.subcore_barrier()

      def g_desc(k, x):
        return pltpu.make_async_copy(z_hbm.at[ebuf.at[0, k]], rows.at[x],
                                     gsem[x])

      def s_desc(k, x):
        return pltpu.make_async_copy(rows.at[x], acc.at[ebuf.at[1, k]],
                                     ssem[x])

      def s_start(k, x):
        pltpu.async_copy(rows.at[x], acc.at[ebuf.at[1, k]], ssem[x],
                         add=True)

      def scale(k, x):
        dnums = lax.GatherDimensionNumbers(
            offset_dims=(), collapsed_slice_dims=(0,), start_index_map=(0,))
        @plsc.parallel_loop(0, SBr // 16, unroll=2)
        def _(g):
          wvec = plsc.bitcast(ebuf[2, k, pl.ds(g * 16, 16)], jnp.float32)
          for j in range(16):
            wj = lax.gather(wvec, jnp.full((16, 1), j, jnp.int32),
                            dimension_numbers=dnums, slice_sizes=(1,),
                            mode=lax.GatherScatterMode.PROMISE_IN_BOUNDS)
            r = g * 16 + j
            for c in range(nvec):
              sl = pl.ds(c * 16, 16)
              rows[x, r, sl] = rows[x, r, sl] * wj

      def phase(p, _):
        @pl.when(p > 0)
        def _():
          for x in (0, 1, 2):
            s_desc(0, x).wait()
        bsl = pl.ds(bbase + p * PB, PB)
        pltpu.async_copy(src_hbm.at[bsl], ebuf.at[0], isem)
        pltpu.async_copy(dst_hbm.at[bsl], ebuf.at[1], isem)
        pltpu.async_copy(w_hbm.at[bsl], ebuf.at[2], isem)
        for q in (0, 1, 2):
          pltpu.make_async_copy(src_hbm.at[bsl], ebuf.at[q], isem).wait()
        g_desc(0, 0).start()

        def step(t, _):
          for x in (0, 1, 2):
            k = 3 * t + x
            y = (x + 1) % 3
            @pl.when(jnp.logical_and(k >= 2, k <= PB - 2))
            def _():
              s_desc(0, y).wait()
            @pl.when(k <= PB - 2)
            def _():
              g_desc(k + 1, y).start()
            g_desc(k, x).wait()
            scale(k, x)
            s_start(k, x)
          return 0
        lax.fori_loop(0, PB // 3, step, 0)
        return 0
      lax.fori_loop(0, nphase, phase, 0)
      for x in (0, 1, 2):
        s_desc(0, x).wait()
      plsc.subcore_barrier()

      sl = pl.ds(sid * ROWS_PT, ROWS_PT)
      pltpu.sync_copy(acc.at[sl], out_hbm.at[sl])

    @pl.when(cid == 0)
    def _():
      run(z0_hbm, out0_hbm)

    @pl.when(cid == 1)
    def _():
      run(z1_hbm, out1_hbm)

  mesh = plsc.VectorSubcoreMesh(core_axis_name="c", subcore_axis_name="s")
  f = pl.kernel(
      body,
      out_type=[jax.ShapeDtypeStruct((N, F), jnp.float32),
                jax.ShapeDtypeStruct((N, F), jnp.float32)],
      mesh=mesh,
      scratch_types=[
          pltpu.VMEM_SHARED((N, F), jnp.float32),
          pltpu.VMEM((3, SBr, F), jnp.float32),
          pltpu.VMEM((3, PB, SBr), jnp.int32),
          pltpu.SemaphoreType.DMA,
          pltpu.SemaphoreType.DMA,
          pltpu.SemaphoreType.DMA,
          pltpu.SemaphoreType.DMA,
          pltpu.SemaphoreType.DMA,
          pltpu.SemaphoreType.DMA,
          pltpu.SemaphoreType.DMA,
      ],
      compiler_params=pltpu.CompilerParams(use_tc_tiling_on_sc=False,
                                           needs_layout_passes=False),
  )
  return f(z0, z1, srcb, dstb, wb)


def _prep_body(x_ref, w1_ref, y_ref, src_ref, dst_ref, ew_ref,
               z0_ref, z1_ref, *sd_ref):
  xw = jnp.dot(x_ref[...], w1_ref[...], preferred_element_type=jnp.float32)
  z0_ref[...] = xw[:, :F1]
  r = xw.shape[0]
  ones = jnp.ones((r, 1), jnp.float32)
  zeros = jnp.zeros((r, F1 - (D_HID - F1) - D_OUT - 1), jnp.float32)
  z1_ref[...] = jnp.concatenate([xw[:, F1:], y_ref[...], ones, zeros], axis=1)

  @pl.when(pl.program_id(0) == 0)
  def _():
    pad = EPAD - E
    ri = lax.broadcasted_iota(jnp.int32, (pad // SB, SB), 0)
    ci = lax.broadcasted_iota(jnp.int32, (pad // SB, SB), 1)
    ar = ((ri * SB + ci) * 16) % N
    sd_ref[0][...] = jnp.concatenate([src_ref[...], ar], axis=0)
    sd_ref[1][...] = jnp.concatenate([dst_ref[...], ar], axis=0)
    wbits = lax.bitcast_convert_type(jnp.exp(ew_ref[...]), jnp.int32)
    zpad = jnp.zeros((pad // SB, SB), jnp.int32)
    sd_ref[2][...] = jnp.concatenate([wbits, zpad], axis=0)


def _mid_body(p0_ref, p1_ref, w2_ref, b1_ref, z2_ref):
  dn = p1_ref[:, D_HID - F1 + D_OUT:D_HID - F1 + D_OUT + 1] + 1e-16
  pre = jnp.concatenate([p0_ref[...], p1_ref[:, :D_HID - F1]], axis=1)
  h = jnp.maximum(pre / dn + b1_ref[...], 0.0)
  hw2 = jnp.dot(h, w2_ref[...], preferred_element_type=jnp.float32)
  z2_ref[...] = jnp.concatenate(
      [hw2, p1_ref[:, D_HID - F1:D_HID - F1 + D_OUT] / dn], axis=1)


def _log_softmax(o):
  o = o - jnp.max(o, axis=1, keepdims=True)
  return o - jnp.log(jnp.sum(jnp.exp(o), axis=1, keepdims=True))


def _final_body(pa_ref, pb_ref, dn_ref, b2_ref, out_ref, y_ref):
  dn = dn_ref[:, D_HID - F1 + D_OUT:D_HID - F1 + D_OUT + 1] + 1e-16
  p2 = pa_ref[...] + pb_ref[...]
  out_ref[...] = _log_softmax(p2[:, :D_OUT] / dn + b2_ref[...])
  y_ref[...] = _log_softmax(p2[:, D_OUT:] / dn)


def kernel(X, adj, Y, W1, b1, W2, b2, edge_weight):
  src = adj[0]
  dst = adj[1]

  R = 1000
  grid = (N // R,)

  z10, z11, srcb, dstb, wb = pl.pallas_call(
      _prep_body,
      grid=grid,
      in_specs=[
          pl.BlockSpec((R, D_IN), lambda i: (i, 0)),
          pl.BlockSpec((D_IN, D_HID), lambda i: (0, 0)),
          pl.BlockSpec((R, D_OUT), lambda i: (i, 0)),
          pl.BlockSpec((E // SB, SB), lambda i: (0, 0)),
          pl.BlockSpec((E // SB, SB), lambda i: (0, 0)),
          pl.BlockSpec((E // SB, SB), lambda i: (0, 0)),
      ],
      out_specs=[
          pl.BlockSpec((R, F1), lambda i: (i, 0)),
          pl.BlockSpec((R, F1), lambda i: (i, 0)),
          pl.BlockSpec((NBTOT, SB), lambda i: (0, 0)),
          pl.BlockSpec((NBTOT, SB), lambda i: (0, 0)),
          pl.BlockSpec((NBTOT, SB), lambda i: (0, 0)),
      ],
      out_shape=[jax.ShapeDtypeStruct((N, F1), jnp.float32),
                 jax.ShapeDtypeStruct((N, F1), jnp.float32),
                 jax.ShapeDtypeStruct((NBTOT, SB), jnp.int32),
                 jax.ShapeDtypeStruct((NBTOT, SB), jnp.int32),
                 jax.ShapeDtypeStruct((NBTOT, SB), jnp.int32)],
  )(X, W1, Y, src.reshape(E // SB, SB), dst.reshape(E // SB, SB),
    edge_weight.reshape(E // SB, SB))

  p10, p11 = _sc_spmm(z10, z11, srcb, dstb, wb, F1, split_edges=False)

  z2 = pl.pallas_call(
      _mid_body,
      grid=grid,
      in_specs=[
          pl.BlockSpec((R, F1), lambda i: (i, 0)),
          pl.BlockSpec((R, F1), lambda i: (i, 0)),
          pl.BlockSpec((D_HID, D_OUT), lambda i: (0, 0)),
          pl.BlockSpec((1, D_HID), lambda i: (0, 0)),
      ],
      out_specs=pl.BlockSpec((R, F2), lambda i: (i, 0)),
      out_shape=jax.ShapeDtypeStruct((N, F2), jnp.float32),
  )(p10, p11, W2, b1.reshape(1, D_HID))

  sb2 = srcb.reshape(EPAD // SB2, SB2)
  db2 = dstb.reshape(EPAD // SB2, SB2)
  wb2 = wb.reshape(EPAD // SB2, SB2)
  p2a, p2b = _sc_spmm(z2, z2, sb2, db2, wb2, F2, split_edges=True)

  out, y2 = pl.pallas_call(
      _final_body,
      grid=grid,
      in_specs=[
          pl.BlockSpec((R, F2), lambda i: (i, 0)),
          pl.BlockSpec((R, F2), lambda i: (i, 0)),
          pl.BlockSpec((R, F1), lambda i: (i, 0)),
          pl.BlockSpec((1, D_OUT), lambda i: (0, 0)),
      ],
      out_specs=[
          pl.BlockSpec((R, D_OUT), lambda i: (i, 0)),
          pl.BlockSpec((R, D_OUT), lambda i: (i, 0)),
      ],
      out_shape=[jax.ShapeDtypeStruct((N, D_OUT), jnp.float32),
                 jax.ShapeDtypeStruct((N, D_OUT), jnp.float32)],
  )(p2a, p2b, p11, b2.reshape(1, D_OUT))

  return (out, y2)


# speedup vs baseline: 1.9363x; 1.0148x over previous
import jax
import jax.numpy as jnp
from jax import lax
from jax.experimental import pallas as pl
from jax.experimental.pallas import tpu as pltpu
from jax.experimental.pallas import tpu_sc as plsc

N = 10000
E = 160000
D_IN = 256
D_HID = 256
D_OUT = 64

F1 = 176
F2 = 128
SB = 32
SB2 = 96
PB1 = 36
PB2 = 27
NSUB = 16
NCORE = 2
EPAD = 165888
NBTOT = EPAD // SB
ROWS_PT = N // NSUB


def _sc_spmm(z0, z1, srcb, dstb, wb, F, split_edges):
  nvec = F // 16
  if split_edges:
    SBr, PB = SB2, PB2
    nb = EPAD // SB2 // (2 * NSUB)
  else:
    SBr, PB = SB, PB1
    nb = NBTOT // NSUB
  nphase = nb // PB

  def body(z0_hbm, z1_hbm, src_hbm, dst_hbm, w_hbm, out0_hbm, out1_hbm,
           acc, rows, ebuf, gs0, gs1, gs2, ss0, ss1, ss2, isem):
    cid = lax.axis_index("c")
    sid = lax.axis_index("s")

    def run(z_hbm, out_hbm):
      if split_edges:
        bbase = (cid * NSUB + sid) * nb
      else:
        bbase = sid * nb
      gsem = (gs0, gs1, gs2)
      ssem = (ss0, ss1, ss2)

      @plsc.parallel_loop(0, SBr)
      def _(j):
        for c in range(nvec):
          rows[0, j, pl.ds(c * 16, 16)] = jnp.zeros((16,), jnp.float32)
      nz = ROWS_PT // SBr
      def zcopy(zi, _):
        pltpu.sync_copy(rows.at[0],
                        acc.at[pl.ds(sid * ROWS_PT + zi * SBr, SBr)])
        return 0
      lax.fori_loop(0, nz, zcopy, 0)
      rem = ROWS_PT - nz * SBr
      if rem:
        pltpu.sync_copy(rows.at[0, pl.ds(0, rem)],
                        acc.at[pl.ds(sid * ROWS_PT + nz * SBr, rem)])
      plsc.subcore_barrier()

      def g_desc(k, x):
        return pltpu.make_async_copy(z_hbm.at[ebuf.at[0, k]], rows.at[x],
                                     gsem[x])

      def s_desc(k, x):
        return pltpu.make_async_copy(rows.at[x], acc.at[ebuf.at[1, k]],
                                     ssem[x])

      def s_start(k, x):
        pltpu.async_copy(rows.at[x], acc.at[ebuf.at[1, k]], ssem[x],
                         add=True)

      def scale(k, x):
        dnums = lax.GatherDimensionNumbers(
            offset_dims=(), collapsed_slice_dims=(0,), start_index_map=(0,))
        @plsc.parallel_loop(0, SBr // 16, unroll=2)
        def _(g):
          wvec = plsc.bitcast(ebuf[2, k, pl.ds(g * 16, 16)], jnp.float32)
          for j in range(16):
            wj = lax.gather(wvec, jnp.full((16, 1), j, jnp.int32),
                            dimension_numbers=dnums, slice_sizes=(1,),
                            mode=lax.GatherScatterMode.PROMISE_IN_BOUNDS)
            r = g * 16 + j
            for c in range(nvec):
              sl = pl.ds(c * 16, 16)
              rows[x, r, sl] = rows[x, r, sl] * wj

      def phase(p, _):
        @pl.when(p > 0)
        def _():
          for x in (0, 1, 2):
            s_desc(0, x).wait()
        bsl = pl.ds(bbase + p * PB, PB)
        pltpu.async_copy(src_hbm.at[bsl], ebuf.at[0], isem)
        pltpu.async_copy(dst_hbm.at[bsl], ebuf.at[1], isem)
        pltpu.async_copy(w_hbm.at[bsl], ebuf.at[2], isem)
        for q in (0, 1, 2):
          pltpu.make_async_copy(src_hbm.at[bsl], ebuf.at[q], isem).wait()
        g_desc(0, 0).start()

        def step(t, _):
          for x in (0, 1, 2):
            k = 3 * t + x
            y = (x + 1) % 3
            @pl.when(jnp.logical_and(k >= 2, k <= PB - 2))
            def _():
              s_desc(0, y).wait()
            @pl.when(k <= PB - 2)
            def _():
              g_desc(k + 1, y).start()
            g_desc(k, x).wait()
            scale(k, x)
            s_start(k, x)
          return 0
        lax.fori_loop(0, PB // 3, step, 0)
        return 0
      lax.fori_loop(0, nphase, phase, 0)
      for x in (0, 1, 2):
        s_desc(0, x).wait()
      plsc.subcore_barrier()

      sl = pl.ds(sid * ROWS_PT, ROWS_PT)
      pltpu.sync_copy(acc.at[sl], out_hbm.at[sl])

    @pl.when(cid == 0)
    def _():
      run(z0_hbm, out0_hbm)

    @pl.when(cid == 1)
    def _():
      run(z1_hbm, out1_hbm)

  mesh = plsc.VectorSubcoreMesh(core_axis_name="c", subcore_axis_name="s")
  f = pl.kernel(
      body,
      out_type=[jax.ShapeDtypeStruct((N, F), jnp.float32),
                jax.ShapeDtypeStruct((N, F), jnp.float32)],
      mesh=mesh,
      scratch_types=[
          pltpu.VMEM_SHARED((N, F), jnp.float32),
          pltpu.VMEM((3, SBr, F), jnp.float32),
          pltpu.VMEM((3, PB, SBr), jnp.int32),
          pltpu.SemaphoreType.DMA,
          pltpu.SemaphoreType.DMA,
          pltpu.SemaphoreType.DMA,
          pltpu.SemaphoreType.DMA,
          pltpu.SemaphoreType.DMA,
          pltpu.SemaphoreType.DMA,
          pltpu.SemaphoreType.DMA,
      ],
      compiler_params=pltpu.CompilerParams(use_tc_tiling_on_sc=False,
                                           needs_layout_passes=False),
  )
  return f(z0, z1, srcb, dstb, wb)


def _prep_body(x_ref, w1_ref, y_ref, src_ref, dst_ref, ew_ref,
               z0_ref, z1_ref, *sd_ref):
  xw = jnp.dot(x_ref[...], w1_ref[...], preferred_element_type=jnp.float32)
  z0_ref[...] = xw[:, :F1]
  r = xw.shape[0]
  ones = jnp.ones((r, 1), jnp.float32)
  zeros = jnp.zeros((r, F1 - (D_HID - F1) - D_OUT - 1), jnp.float32)
  z1_ref[...] = jnp.concatenate([xw[:, F1:], y_ref[...], ones, zeros], axis=1)

  @pl.when(pl.program_id(0) == 0)
  def _():
    pad = EPAD - E
    ri = lax.broadcasted_iota(jnp.int32, (pad // SB, SB), 0)
    ci = lax.broadcasted_iota(jnp.int32, (pad // SB, SB), 1)
    ar = ((ri * SB + ci) * 16) % N
    sd_ref[0][...] = jnp.concatenate([src_ref[...], ar], axis=0)
    sd_ref[1][...] = jnp.concatenate([dst_ref[...], ar], axis=0)
    wbits = lax.bitcast_convert_type(jnp.exp(ew_ref[...]), jnp.int32)
    zpad = jnp.zeros((pad // SB, SB), jnp.int32)
    sd_ref[2][...] = jnp.concatenate([wbits, zpad], axis=0)


def _mid_body(p0_ref, p1_ref, w2_ref, b1_ref, z2_ref):
  dn = p1_ref[:, D_HID - F1 + D_OUT:D_HID - F1 + D_OUT + 1] + 1e-16
  pre = jnp.concatenate([p0_ref[...], p1_ref[:, :D_HID - F1]], axis=1)
  h = jnp.maximum(pre / dn + b1_ref[...], 0.0)
  hw2 = jnp.dot(h, w2_ref[...], preferred_element_type=jnp.float32)
  z2_ref[...] = jnp.concatenate(
      [hw2, p1_ref[:, D_HID - F1:D_HID - F1 + D_OUT] / dn], axis=1)


def _log_softmax(o):
  o = o - jnp.max(o, axis=1, keepdims=True)
  return o - jnp.log(jnp.sum(jnp.exp(o), axis=1, keepdims=True))


def _final_body(pa_ref, pb_ref, dn_ref, b2_ref, out_ref, y_ref):
  dn = dn_ref[:, D_HID - F1 + D_OUT:D_HID - F1 + D_OUT + 1] + 1e-16
  p2 = pa_ref[...] + pb_ref[...]
  out_ref[...] = _log_softmax(p2[:, :D_OUT] / dn + b2_ref[...])
  y_ref[...] = _log_softmax(p2[:, D_OUT:] / dn)


def kernel(X, adj, Y, W1, b1, W2, b2, edge_weight):
  src = adj[0]
  dst = adj[1]

  R = 2000
  grid = (N // R,)

  z10, z11, srcb, dstb, wb = pl.pallas_call(
      _prep_body,
      grid=grid,
      in_specs=[
          pl.BlockSpec((R, D_IN), lambda i: (i, 0)),
          pl.BlockSpec((D_IN, D_HID), lambda i: (0, 0)),
          pl.BlockSpec((R, D_OUT), lambda i: (i, 0)),
          pl.BlockSpec((E // SB, SB), lambda i: (0, 0)),
          pl.BlockSpec((E // SB, SB), lambda i: (0, 0)),
          pl.BlockSpec((E // SB, SB), lambda i: (0, 0)),
      ],
      out_specs=[
          pl.BlockSpec((R, F1), lambda i: (i, 0)),
          pl.BlockSpec((R, F1), lambda i: (i, 0)),
          pl.BlockSpec((NBTOT, SB), lambda i: (0, 0)),
          pl.BlockSpec((NBTOT, SB), lambda i: (0, 0)),
          pl.BlockSpec((NBTOT, SB), lambda i: (0, 0)),
      ],
      out_shape=[jax.ShapeDtypeStruct((N, F1), jnp.float32),
                 jax.ShapeDtypeStruct((N, F1), jnp.float32),
                 jax.ShapeDtypeStruct((NBTOT, SB), jnp.int32),
                 jax.ShapeDtypeStruct((NBTOT, SB), jnp.int32),
                 jax.ShapeDtypeStruct((NBTOT, SB), jnp.int32)],
  )(X, W1, Y, src.reshape(E // SB, SB), dst.reshape(E // SB, SB),
    edge_weight.reshape(E // SB, SB))

  p10, p11 = _sc_spmm(z10, z11, srcb, dstb, wb, F1, split_edges=False)

  z2 = pl.pallas_call(
      _mid_body,
      grid=grid,
      in_specs=[
          pl.BlockSpec((R, F1), lambda i: (i, 0)),
          pl.BlockSpec((R, F1), lambda i: (i, 0)),
          pl.BlockSpec((D_HID, D_OUT), lambda i: (0, 0)),
          pl.BlockSpec((1, D_HID), lambda i: (0, 0)),
      ],
      out_specs=pl.BlockSpec((R, F2), lambda i: (i, 0)),
      out_shape=jax.ShapeDtypeStruct((N, F2), jnp.float32),
  )(p10, p11, W2, b1.reshape(1, D_HID))

  sb2 = srcb.reshape(EPAD // SB2, SB2)
  db2 = dstb.reshape(EPAD // SB2, SB2)
  wb2 = wb.reshape(EPAD // SB2, SB2)
  p2a, p2b = _sc_spmm(z2, z2, sb2, db2, wb2, F2, split_edges=True)

  out, y2 = pl.pallas_call(
      _final_body,
      grid=grid,
      in_specs=[
          pl.BlockSpec((R, F2), lambda i: (i, 0)),
          pl.BlockSpec((R, F2), lambda i: (i, 0)),
          pl.BlockSpec((R, F1), lambda i: (i, 0)),
          pl.BlockSpec((1, D_OUT), lambda i: (0, 0)),
      ],
      out_specs=[
          pl.BlockSpec((R, D_OUT), lambda i: (i, 0)),
          pl.BlockSpec((R, D_OUT), lambda i: (i, 0)),
      ],
      out_shape=[jax.ShapeDtypeStruct((N, D_OUT), jnp.float32),
                 jax.ShapeDtypeStruct((N, D_OUT), jnp.float32)],
  )(p2a, p2b, p11, b2.reshape(1, D_OUT))

  return (out, y2)
